# baseline (speedup 1.0000x reference)
import functools

import jax
import jax.numpy as jnp
from jax import lax
from jax.experimental import pallas as pl
from jax.experimental.pallas import tpu as pltpu
from jax.experimental.pallas import tpu_sc as plsc

D_MODEL = 768
MAX_LEN = 2048
SEQ = 512
BASE = MAX_LEN - SEQ
WIN = 1024

NUM_CORES = 2
NUM_SUBCORES = 16
NUM_WORKERS = NUM_CORES * NUM_SUBCORES
ROWS_PER_WORKER = SEQ // NUM_WORKERS


@functools.partial(
    pl.kernel,
    mesh=plsc.VectorSubcoreMesh(core_axis_name="c", subcore_axis_name="s"),
    out_type=jax.ShapeDtypeStruct((SEQ, SEQ, D_MODEL), jnp.float32),
    scratch_types=[pltpu.SemaphoreType.DMA],
)
def _rpe_sc(shifted_hbm, out_hbm, sem):
    c = lax.axis_index("c")
    s = lax.axis_index("s")
    wid = s * NUM_CORES + c
    base = wid * ROWS_PER_WORKER
    copies = []
    for r in range(ROWS_PER_WORKER):
        i = base + r
        a = (MAX_LEN - 1) - i
        p = lax.rem(a, 8)
        off = pl.multiple_of(a - BASE - p, 8)
        copies.append(
            pltpu.async_copy(
                shifted_hbm.at[p, pl.ds(off, SEQ)], out_hbm.at[i], sem
            )
        )
    for cp in copies:
        cp.wait()


def kernel(rel_emb, length):
    del length
    shifted = jnp.stack(
        [lax.slice_in_dim(rel_emb, BASE + p, BASE + p + WIN, axis=0)
         for p in range(8)]
    )
    return _rpe_sc(shifted)


# speedup vs baseline: 42.4975x; 42.4975x over previous
import functools

import jax
import jax.numpy as jnp
from jax import lax
from jax.experimental import pallas as pl
from jax.experimental.pallas import tpu as pltpu
from jax.experimental.pallas import tpu_sc as plsc

D_MODEL = 768
MAX_LEN = 2048
SEQ = 512
BASE = MAX_LEN - SEQ
WIN = 1024

NUM_CORES = 2
NUM_SUBCORES = 16
NUM_WORKERS = NUM_CORES * NUM_SUBCORES
ROWS_PER_WORKER = SEQ // NUM_WORKERS
CHUNK = 128
CHUNKS_PER_ROW = SEQ // CHUNK


@functools.partial(
    pl.kernel,
    mesh=plsc.VectorSubcoreMesh(core_axis_name="c", subcore_axis_name="s"),
    out_type=jax.ShapeDtypeStruct((SEQ, SEQ, D_MODEL), jnp.float32),
    scratch_types=[
        pltpu.VMEM_SHARED((NUM_SUBCORES, CHUNK, D_MODEL), jnp.float32),
        pltpu.SemaphoreType.DMA,
    ],
)
def _rpe_sc(shifted_hbm, out_hbm, buf, sem):
    c = lax.axis_index("c")
    s = lax.axis_index("s")
    wid = s * NUM_CORES + c
    base = wid * ROWS_PER_WORKER
    for r in range(ROWS_PER_WORKER):
        i = base + r
        a = (MAX_LEN - 1) - i
        p = lax.rem(a, 8)
        off = pl.multiple_of(a - BASE - p, 8)
        for kc in range(CHUNKS_PER_ROW):
            j0 = kc * CHUNK
            pltpu.sync_copy(shifted_hbm.at[p, pl.ds(off + j0, CHUNK)], buf.at[s])
            pltpu.sync_copy(buf.at[s], out_hbm.at[i, pl.ds(j0, CHUNK)])


def kernel(rel_emb, length):
    del length
    shifted = jnp.stack(
        [lax.slice_in_dim(rel_emb, BASE + p, BASE + p + WIN, axis=0)
         for p in range(8)]
    )
    return _rpe_sc(shifted)


# speedup vs baseline: 42.6180x; 1.0028x over previous
import functools

import jax
import jax.numpy as jnp
from jax import lax
from jax.experimental import pallas as pl
from jax.experimental.pallas import tpu as pltpu
from jax.experimental.pallas import tpu_sc as plsc

D_MODEL = 768
MAX_LEN = 2048
SEQ = 512
BASE = MAX_LEN - SEQ
WIN = 1024

NUM_CORES = 2
NUM_SUBCORES = 16
NUM_WORKERS = NUM_CORES * NUM_SUBCORES
ROWS_PER_WORKER = SEQ // NUM_WORKERS
CHUNK = 64
CHUNKS_PER_ROW = SEQ // CHUNK
NBUF = 2
NCHUNKS = ROWS_PER_WORKER * CHUNKS_PER_ROW


@functools.partial(
    pl.kernel,
    mesh=plsc.VectorSubcoreMesh(core_axis_name="c", subcore_axis_name="s"),
    out_type=jax.ShapeDtypeStruct((SEQ, SEQ, D_MODEL), jnp.float32),
    scratch_types=[
        pltpu.VMEM_SHARED((NUM_SUBCORES, NBUF, CHUNK, D_MODEL), jnp.float32),
        pltpu.SemaphoreType.DMA,
        pltpu.SemaphoreType.DMA,
        pltpu.SemaphoreType.DMA,
        pltpu.SemaphoreType.DMA,
    ],
)
def _rpe_sc(shifted_hbm, out_hbm, buf, sem_in0, sem_in1, sem_out0, sem_out1):
    sem_in = [sem_in0, sem_in1]
    sem_out = [sem_out0, sem_out1]
    c = lax.axis_index("c")
    s = lax.axis_index("s")
    wid = s * NUM_CORES + c
    base = wid * ROWS_PER_WORKER

    def src_chunk(k):
        i = base + (k // CHUNKS_PER_ROW)
        a = (MAX_LEN - 1) - i
        p = lax.rem(a, 8)
        off = pl.multiple_of(a - BASE - p, 8)
        j0 = (k % CHUNKS_PER_ROW) * CHUNK
        return shifted_hbm.at[p, pl.ds(off + j0, CHUNK)]

    def dst_chunk(k):
        i = base + (k // CHUNKS_PER_ROW)
        j0 = (k % CHUNKS_PER_ROW) * CHUNK
        return out_hbm.at[i, pl.ds(j0, CHUNK)]

    in_cp = [None] * NBUF
    out_cp = [None] * NBUF
    for k in range(NCHUNKS + 1):
        if k < NCHUNKS:
            b = k % NBUF
            if out_cp[b] is not None:
                out_cp[b].wait()
            in_cp[b] = pltpu.async_copy(src_chunk(k), buf.at[s, b], sem_in[b])
        if k >= 1:
            kb = (k - 1) % NBUF
            in_cp[kb].wait()
            out_cp[kb] = pltpu.async_copy(buf.at[s, kb], dst_chunk(k - 1), sem_out[kb])
    for b in range(NBUF):
        if out_cp[b] is not None:
            out_cp[b].wait()


def kernel(rel_emb, length):
    del length
    shifted = jnp.stack(
        [lax.slice_in_dim(rel_emb, BASE + p, BASE + p + WIN, axis=0)
         for p in range(8)]
    )
    return _rpe_sc(shifted)


# speedup vs baseline: 45.3225x; 1.0635x over previous
import functools

import jax
import jax.numpy as jnp
from jax import lax
from jax.experimental import pallas as pl
from jax.experimental.pallas import tpu as pltpu
from jax.experimental.pallas import tpu_sc as plsc

D_MODEL = 768
MAX_LEN = 2048
SEQ = 512
BASE = MAX_LEN - SEQ
WIN = 1024

NUM_CORES = 2
NUM_SUBCORES = 16
STAGE_ROWS = WIN // NUM_SUBCORES
CHUNK = 32
CHUNKS_PER_ROW = SEQ // CHUNK


@functools.partial(
    pl.kernel,
    mesh=plsc.VectorSubcoreMesh(core_axis_name="c", subcore_axis_name="s"),
    out_type=jax.ShapeDtypeStruct((SEQ, SEQ, D_MODEL), jnp.float32),
    scratch_types=[
        pltpu.VMEM_SHARED((2, WIN, D_MODEL), jnp.float32),
        pltpu.VMEM_SHARED((NUM_SUBCORES, CHUNK, D_MODEL), jnp.float32),
        pltpu.SemaphoreType.DMA,
    ],
)
def _rpe_sc(shifted_hbm, out_hbm, stage, buf, sem):
    c = lax.axis_index("c")
    s = lax.axis_index("s")

    for u in range(2):
        pltpu.sync_copy(
            shifted_hbm.at[2 * c + u, pl.ds(s * STAGE_ROWS, STAGE_ROWS)],
            stage.at[u, pl.ds(s * STAGE_ROWS, STAGE_ROWS)],
        )
    plsc.subcore_barrier()

    copies = []
    for q in range(4):
        m = s * 4 + q
        off = pl.multiple_of(504 - 8 * m, 8)
        for u in range(2):
            i = 8 * m + (7 - u) - 2 * c
            copies.append(
                pltpu.async_copy(stage.at[u, pl.ds(off, SEQ)], out_hbm.at[i], sem)
            )

    for q in range(4):
        m = s * 4 + q
        off = pl.multiple_of(504 - 8 * m, 8)
        for u in range(2):
            i = 8 * m + (3 - u) - 2 * c
            p = (4 + u) + 2 * c
            for kc in range(CHUNKS_PER_ROW):
                j0 = kc * CHUNK
                pltpu.sync_copy(shifted_hbm.at[p, pl.ds(off + j0, CHUNK)],
                                buf.at[s])
                pltpu.sync_copy(buf.at[s], out_hbm.at[i, pl.ds(j0, CHUNK)])

    for cp in copies:
        cp.wait()


def kernel(rel_emb, length):
    del length
    shifted = jnp.stack(
        [lax.slice_in_dim(rel_emb, BASE + p, BASE + p + WIN, axis=0)
         for p in range(8)]
    )
    return _rpe_sc(shifted)


# speedup vs baseline: 48.6387x; 1.0732x over previous
import functools

import jax
import jax.numpy as jnp
from jax import lax
from jax.experimental import pallas as pl
from jax.experimental.pallas import tpu as pltpu
from jax.experimental.pallas import tpu_sc as plsc

D_MODEL = 768
MAX_LEN = 2048
SEQ = 512
BASE = MAX_LEN - SEQ
WIN = 1024

NUM_CORES = 2
NUM_SUBCORES = 16
STAGE_ROWS = WIN // NUM_SUBCORES
CHUNK = 32
CHUNKS_PER_ROW = SEQ // CHUNK


@functools.partial(
    pl.kernel,
    mesh=plsc.VectorSubcoreMesh(core_axis_name="c", subcore_axis_name="s"),
    out_type=jax.ShapeDtypeStruct((SEQ, SEQ, D_MODEL), jnp.float32),
    scratch_types=[
        pltpu.VMEM_SHARED((2, WIN, D_MODEL), jnp.float32),
        pltpu.VMEM((CHUNK, D_MODEL), jnp.float32),
        pltpu.SemaphoreType.DMA,
    ],
)
def _rpe_sc(shifted_hbm, out_hbm, stage, buf, sem):
    c = lax.axis_index("c")
    s = lax.axis_index("s")

    for u in range(2):
        pltpu.sync_copy(
            shifted_hbm.at[2 * c + u, pl.ds(s * STAGE_ROWS, STAGE_ROWS)],
            stage.at[u, pl.ds(s * STAGE_ROWS, STAGE_ROWS)],
        )
    plsc.subcore_barrier()

    copies = []
    for q in range(4):
        m = s * 4 + q
        off = pl.multiple_of(504 - 8 * m, 8)
        for u in range(2):
            i = 8 * m + (7 - u) - 2 * c
            copies.append(
                pltpu.async_copy(stage.at[u, pl.ds(off, SEQ)], out_hbm.at[i], sem)
            )

    for q in range(4):
        m = s * 4 + q
        off = pl.multiple_of(504 - 8 * m, 8)
        for u in range(2):
            i = 8 * m + (3 - u) - 2 * c
            p = (4 + u) + 2 * c
            for kc in range(CHUNKS_PER_ROW):
                j0 = kc * CHUNK
                pltpu.sync_copy(shifted_hbm.at[p, pl.ds(off + j0, CHUNK)], buf)
                pltpu.sync_copy(buf, out_hbm.at[i, pl.ds(j0, CHUNK)])

    for cp in copies:
        cp.wait()


def kernel(rel_emb, length):
    del length
    shifted = jnp.stack(
        [lax.slice_in_dim(rel_emb, BASE + p, BASE + p + WIN, axis=0)
         for p in range(8)]
    )
    return _rpe_sc(shifted)
